# baseline (speedup 1.0000x reference)
import functools

import jax
import jax.numpy as jnp
from jax import lax
from jax.experimental import pallas as pl
from jax.experimental.pallas import tpu as pltpu
from jax.experimental.pallas import tpu_sc as plsc

_IN = 2048
_OUT = 8192
_RANK = 2
_B = 4096
_K = _RANK * _OUT
_NC = 2
_NS = 16
_NW = _NC * _NS
_RW = _B // _NW
_L = 16
_UNROLL = 8


@functools.partial(
    pl.kernel,
    out_type=jax.ShapeDtypeStruct((_B * _K,), jnp.float32),
    mesh=plsc.VectorSubcoreMesh(core_axis_name="c", subcore_axis_name="s",
                                num_cores=_NC),
    scratch_types=[
        pltpu.VMEM((_K,), jnp.int32),
        pltpu.VMEM((_IN,), jnp.float32),
        pltpu.VMEM((_K,), jnp.float32),
    ],
    compiler_params=pltpu.CompilerParams(needs_layout_passes=False),
)
def _lut_gather(x_hbm, idx_hbm, out_hbm, idx_v, row_v, out_v):
    wid = lax.axis_index("s") * _NC + lax.axis_index("c")
    base = wid * _RW
    pltpu.sync_copy(idx_hbm, idx_v)

    def do_row(i, carry):
        b = base + i
        pltpu.sync_copy(x_hbm.at[pl.ds(b * _IN, _IN)], row_v)

        def do_chunk(j, c2):
            k0 = pl.multiple_of(j * (_L * _UNROLL), _L * _UNROLL)
            for u in range(_UNROLL):
                off = k0 + u * _L
                idxv = idx_v[pl.ds(off, _L)]
                out_v[pl.ds(off, _L)] = plsc.load_gather(row_v, [idxv])
            return c2

        lax.fori_loop(0, _K // (_L * _UNROLL), do_chunk, 0)
        pltpu.sync_copy(out_v, out_hbm.at[pl.ds(b * _K, _K)])
        return carry

    lax.fori_loop(0, _RW, do_row, 0)


@jax.jit
def kernel(x, indices):
    idx = indices.reshape(-1).astype(jnp.int32)
    out = _lut_gather(x.reshape(-1), idx)
    return out.reshape(_B, _RANK, _OUT)


# speedup vs baseline: 2.1972x; 2.1972x over previous
import functools

import jax
import jax.numpy as jnp
from jax import lax
from jax.experimental import pallas as pl
from jax.experimental.pallas import tpu as pltpu
from jax.experimental.pallas import tpu_sc as plsc

_IN = 2048
_OUT = 8192
_RANK = 2
_B = 4096
_K = _RANK * _OUT
_NC = 2
_NS = 16
_NW = _NC * _NS
_RW = _B // _NW
_L = 16
_R = 2
_G = _RW // _R


@functools.partial(
    pl.kernel,
    out_type=jax.ShapeDtypeStruct((_B * _K,), jnp.float32),
    mesh=plsc.VectorSubcoreMesh(core_axis_name="c", subcore_axis_name="s",
                                num_cores=_NC),
    scratch_types=[
        pltpu.VMEM((_K,), jnp.int32),
        pltpu.VMEM((_R * _IN,), jnp.float32),
        pltpu.VMEM((_R * _IN,), jnp.float32),
        pltpu.VMEM((_R * _K,), jnp.float32),
        pltpu.VMEM((_R * _K,), jnp.float32),
        pltpu.SemaphoreType.DMA,
        pltpu.SemaphoreType.DMA,
        pltpu.SemaphoreType.DMA,
        pltpu.SemaphoreType.DMA,
    ],
    compiler_params=pltpu.CompilerParams(needs_layout_passes=False),
)
def _lut_gather(x_hbm, idx_hbm, out_hbm, idx_v, x0, x1, o0, o1,
                si0, si1, so0, so1):
    wid = lax.axis_index("s") * _NC + lax.axis_index("c")
    base = wid * _RW
    pltpu.sync_copy(idx_hbm, idx_v)

    xb, ob, si, so = (x0, x1), (o0, o1), (si0, si1), (so0, so1)

    def in_copy(g, buf, sem):
        return pltpu.make_async_copy(
            x_hbm.at[pl.ds((base + g * _R) * _IN, _R * _IN)], buf, sem)

    def out_copy(g, buf, sem):
        return pltpu.make_async_copy(
            buf, out_hbm.at[pl.ds((base + g * _R) * _K, _R * _K)], sem)

    in_copy(0, x0, si0).start()

    def grp_pair(gp, carry):
        for par in range(2):
            g = gp * 2 + par
            xv, ov = xb[par], ob[par]
            in_copy(g, xv, si[par]).wait()

            @pl.when(g + 1 < _G)
            def _start_next():
                in_copy(g + 1, xb[1 - par], si[1 - par]).start()

            @pl.when(g >= 2)
            def _wait_out():
                out_copy(g - 2, ov, so[par]).wait()

            @plsc.parallel_loop(0, _K, step=_L, unroll=8)
            def chunk(off):
                iv = idx_v[pl.ds(off, _L)]
                ov[pl.ds(off, _L)] = plsc.load_gather(xv, [iv])
                ov[pl.ds(_K + off, _L)] = plsc.load_gather(xv, [iv + _IN])

            out_copy(g, ov, so[par]).start()
        return carry

    lax.fori_loop(0, _G // 2, grp_pair, 0)
    out_copy(_G - 2, o0, so0).wait()
    out_copy(_G - 1, o1, so1).wait()


@jax.jit
def kernel(x, indices):
    idx = indices.reshape(-1).astype(jnp.int32)
    out = _lut_gather(x.reshape(-1), idx)
    return out.reshape(_B, _RANK, _OUT)


# speedup vs baseline: 6.5064x; 2.9612x over previous
import functools

import jax
import jax.numpy as jnp
from jax import lax
from jax.experimental import pallas as pl
from jax.experimental.pallas import tpu as pltpu
from jax.experimental.pallas import tpu_sc as plsc

_IN = 2048
_OUT = 8192
_RANK = 2
_B = 4096
_K = _RANK * _OUT
_NC = 2
_NS = 16
_NW = _NC * _NS
_RW = _B // _NW
_L = 16
_R = 2
_G = _RW // _R


@functools.partial(
    pl.kernel,
    out_type=jax.ShapeDtypeStruct((_B, _RANK, _OUT), jnp.float32),
    mesh=plsc.VectorSubcoreMesh(core_axis_name="c", subcore_axis_name="s",
                                num_cores=_NC),
    scratch_types=[
        pltpu.VMEM((_K,), jnp.int32),
        pltpu.VMEM((_R * _IN,), jnp.float32),
        pltpu.VMEM((_R * _IN,), jnp.float32),
        pltpu.VMEM((_R * _K,), jnp.float32),
        pltpu.VMEM((_R * _K,), jnp.float32),
        pltpu.SemaphoreType.DMA,
        pltpu.SemaphoreType.DMA,
        pltpu.SemaphoreType.DMA,
        pltpu.SemaphoreType.DMA,
    ],
    compiler_params=pltpu.CompilerParams(needs_layout_passes=False),
)
def _lut_gather(x_hbm, idx_hbm, out_hbm, idx_v, x0, x1, o0, o1,
                si0, si1, so0, so1):
    wid = lax.axis_index("s") * _NC + lax.axis_index("c")
    base = wid * _RW
    pltpu.sync_copy(idx_hbm, idx_v)

    xb, ob, si, so = (x0, x1), (o0, o1), (si0, si1), (so0, so1)

    def in_copies(g, buf, sem):
        row0 = base + g * _R
        return [
            pltpu.make_async_copy(
                x_hbm.at[row0 + q], buf.at[pl.ds(q * _IN, _IN)], sem)
            for q in range(_R)
        ]

    def out_copies(g, buf, sem):
        row0 = base + g * _R
        return [
            pltpu.make_async_copy(
                buf.at[pl.ds(q * _K + r * _OUT, _OUT)],
                out_hbm.at[row0 + q, r], sem)
            for q in range(_R)
            for r in range(_RANK)
        ]

    for c in in_copies(0, x0, si0):
        c.start()

    def grp_pair(gp, carry):
        for par in range(2):
            g = gp * 2 + par
            xv, ov = xb[par], ob[par]
            for c in in_copies(g, xv, si[par]):
                c.wait()

            @pl.when(g + 1 < _G)
            def _start_next():
                for c in in_copies(g + 1, xb[1 - par], si[1 - par]):
                    c.start()

            @pl.when(g >= 2)
            def _wait_out():
                for c in out_copies(g - 2, ov, so[par]):
                    c.wait()

            @plsc.parallel_loop(0, _K, step=_L, unroll=8)
            def chunk(off):
                iv = idx_v[pl.ds(off, _L)]
                ov[pl.ds(off, _L)] = plsc.load_gather(xv, [iv])
                ov[pl.ds(_K + off, _L)] = plsc.load_gather(xv, [iv + _IN])

            for c in out_copies(g, ov, so[par]):
                c.start()
        return carry

    lax.fori_loop(0, _G // 2, grp_pair, 0)
    for c in out_copies(_G - 2, o0, so0):
        c.wait()
    for c in out_copies(_G - 1, o1, so1):
        c.wait()


@jax.jit
def kernel(x, indices):
    idx = indices.reshape(-1).astype(jnp.int32)
    return _lut_gather(x, idx)


# speedup vs baseline: 7.2051x; 1.1074x over previous
import functools

import jax
import jax.numpy as jnp
from jax import lax
from jax.experimental import pallas as pl
from jax.experimental.pallas import tpu as pltpu
from jax.experimental.pallas import tpu_sc as plsc

_IN = 2048
_OUT = 8192
_RANK = 2
_B = 4096
_K = _RANK * _OUT
_NC = 2
_NS = 16
_NW = _NC * _NS
_RW = _B // _NW
_L = 16
_R = 8
_G = _RW // _R
_SEG = 2048
_NSEG = _K // _SEG
_SPR = _OUT // _SEG


@functools.partial(
    pl.kernel,
    out_type=jax.ShapeDtypeStruct((_B, _RANK, _OUT), jnp.float32),
    mesh=plsc.VectorSubcoreMesh(core_axis_name="c", subcore_axis_name="s",
                                num_cores=_NC),
    scratch_types=[
        pltpu.VMEM((_K,), jnp.int32),
        pltpu.VMEM((_R * _IN,), jnp.float32),
        pltpu.VMEM((_R * _IN,), jnp.float32),
        pltpu.VMEM((_R * _SEG,), jnp.float32),
        pltpu.VMEM((_R * _SEG,), jnp.float32),
        pltpu.SemaphoreType.DMA,
        pltpu.SemaphoreType.DMA,
        pltpu.SemaphoreType.DMA,
        pltpu.SemaphoreType.DMA,
    ],
    compiler_params=pltpu.CompilerParams(needs_layout_passes=False),
)
def _lut_gather(x_hbm, idx_hbm, out_hbm, idx_v, x0, x1, o0, o1,
                si0, si1, so0, so1):
    wid = lax.axis_index("s") * _NC + lax.axis_index("c")
    base = wid * _RW
    pltpu.sync_copy(idx_hbm, idx_v)

    xb, ob, si, so = (x0, x1), (o0, o1), (si0, si1), (so0, so1)

    def in_copies(g, buf, sem):
        row0 = base + g * _R
        return [
            pltpu.make_async_copy(
                x_hbm.at[row0 + q], buf.at[pl.ds(q * _IN, _IN)], sem)
            for q in range(_R)
        ]

    def seg_copies(g, s, buf, sem):
        row0 = base + g * _R
        return [
            pltpu.make_async_copy(
                buf.at[pl.ds(q * _SEG, _SEG)],
                out_hbm.at[row0 + q, s // _SPR,
                           pl.ds((s % _SPR) * _SEG, _SEG)],
                sem)
            for q in range(_R)
        ]

    for c in in_copies(0, x0, si0):
        c.start()

    def do_group(g, carry):
        gpar = lax.rem(g, 2)

        def run_parity(par):
            xv = xb[par]
            for c in in_copies(g, xv, si[par]):
                c.wait()

            @pl.when(g + 1 < _G)
            def _start_next():
                for c in in_copies(g + 1, xb[1 - par], si[1 - par]):
                    c.start()

            for s in range(_NSEG):
                sp = s % 2
                ov = ob[sp]
                if s >= 2:
                    for c in seg_copies(g, s, ov, so[sp]):
                        c.wait()
                else:
                    @pl.when(g >= 1)
                    def _wait_prev_group():
                        for c in seg_copies(g, s, ov, so[sp]):
                            c.wait()

                @plsc.parallel_loop(0, _SEG, step=_L, unroll=4)
                def chunk(off):
                    iv = idx_v[pl.ds(s * _SEG + off, _L)]
                    for q in range(_R):
                        ov[pl.ds(q * _SEG + off, _L)] = (
                            plsc.load_gather(xv, [iv + q * _IN]))

                for c in seg_copies(g, s, ov, so[sp]):
                    c.start()

        @pl.when(gpar == 0)
        def _p0():
            run_parity(0)

        @pl.when(gpar == 1)
        def _p1():
            run_parity(1)

        return carry

    lax.fori_loop(0, _G, do_group, 0)
    for c in seg_copies(_G - 1, _NSEG - 2, o0, so0):
        c.wait()
    for c in seg_copies(_G - 1, _NSEG - 1, o1, so1):
        c.wait()


@jax.jit
def kernel(x, indices):
    idx = indices.reshape(-1).astype(jnp.int32)
    return _lut_gather(x, idx)


# speedup vs baseline: 7.4601x; 1.0354x over previous
import functools

import jax
import jax.numpy as jnp
from jax import lax
from jax.experimental import pallas as pl
from jax.experimental.pallas import tpu as pltpu
from jax.experimental.pallas import tpu_sc as plsc

_IN = 2048
_OUT = 8192
_RANK = 2
_B = 4096
_K = _RANK * _OUT
_NC = 2
_NS = 16
_NW = _NC * _NS
_RW = _B // _NW
_L = 16
_R = 8
_G = _RW // _R
_SEG = 4096
_NSEG = _K // _SEG
_SPR = _OUT // _SEG


@functools.partial(
    pl.kernel,
    out_type=jax.ShapeDtypeStruct((_B, _RANK, _OUT), jnp.float32),
    mesh=plsc.VectorSubcoreMesh(core_axis_name="c", subcore_axis_name="s",
                                num_cores=_NC),
    scratch_types=[
        pltpu.VMEM((_K,), jnp.int32),
        pltpu.VMEM((_R * _IN,), jnp.float32),
        pltpu.VMEM((_R * _IN,), jnp.float32),
        pltpu.VMEM((_R * _SEG,), jnp.float32),
        pltpu.VMEM((_R * _SEG,), jnp.float32),
        pltpu.SemaphoreType.DMA,
        pltpu.SemaphoreType.DMA,
        pltpu.SemaphoreType.DMA,
        pltpu.SemaphoreType.DMA,
    ],
    compiler_params=pltpu.CompilerParams(needs_layout_passes=False),
)
def _lut_gather(x_hbm, idx_hbm, out_hbm, idx_v, x0, x1, o0, o1,
                si0, si1, so0, so1):
    wid = lax.axis_index("s") * _NC + lax.axis_index("c")
    base = wid * _RW
    for r in range(_RANK):
        pltpu.sync_copy(idx_hbm.at[r], idx_v.at[pl.ds(r * _OUT, _OUT)])

    xb, ob, si, so = (x0, x1), (o0, o1), (si0, si1), (so0, so1)

    def in_copies(g, buf, sem):
        row0 = base + g * _R
        return [
            pltpu.make_async_copy(
                x_hbm.at[row0 + q], buf.at[pl.ds(q * _IN, _IN)], sem)
            for q in range(_R)
        ]

    def seg_copies(g, s, buf, sem):
        row0 = base + g * _R
        return [
            pltpu.make_async_copy(
                buf.at[pl.ds(q * _SEG, _SEG)],
                out_hbm.at[row0 + q, s // _SPR,
                           pl.ds((s % _SPR) * _SEG, _SEG)],
                sem)
            for q in range(_R)
        ]

    for c in in_copies(0, x0, si0):
        c.start()

    def do_group(g, carry):
        gpar = lax.rem(g, 2)

        def run_parity(par):
            xv = xb[par]
            for c in in_copies(g, xv, si[par]):
                c.wait()

            @pl.when(g + 1 < _G)
            def _start_next():
                for c in in_copies(g + 1, xb[1 - par], si[1 - par]):
                    c.start()

            for s in range(_NSEG):
                sp = s % 2
                ov = ob[sp]
                if s >= 2:
                    for c in seg_copies(g, s, ov, so[sp]):
                        c.wait()
                else:
                    @pl.when(g >= 1)
                    def _wait_prev_group():
                        for c in seg_copies(g, s, ov, so[sp]):
                            c.wait()

                @plsc.parallel_loop(0, _SEG, step=_L, unroll=8)
                def chunk(off):
                    iv = idx_v[pl.ds(s * _SEG + off, _L)]
                    for q in range(_R):
                        ov[pl.ds(q * _SEG + off, _L)] = (
                            plsc.load_gather(xv, [iv + q * _IN]))

                for c in seg_copies(g, s, ov, so[sp]):
                    c.start()

        @pl.when(gpar == 0)
        def _p0():
            run_parity(0)

        @pl.when(gpar == 1)
        def _p1():
            run_parity(1)

        return carry

    lax.fori_loop(0, _G, do_group, 0)
    for c in seg_copies(_G - 1, _NSEG - 2, o0, so0):
        c.wait()
    for c in seg_copies(_G - 1, _NSEG - 1, o1, so1):
        c.wait()


@jax.jit
def kernel(x, indices):
    return _lut_gather(x, indices.astype(jnp.int32))
